# Initial kernel scaffold; baseline (speedup 1.0000x reference)
import numpy as np
import jax
import jax.numpy as jnp
from jax.experimental import pallas as pl

_N = 4096
_D = 64
_K = 4
_S = _K - 1
_RA = 256
_RB = 128
_CUTOFF = 0.5
_NLC = 1.4
_TINY = np.float32(np.finfo(np.float32).tiny)


def _np_threefry2x32(k0, k1, x0, x1):
    ks = (np.uint32(k0), np.uint32(k1),
          np.uint32(k0) ^ np.uint32(k1) ^ np.uint32(0x1BD11BDA))
    rots = ((13, 15, 26, 6), (17, 29, 16, 24))
    x0 = (x0 + ks[0]).astype(np.uint32)
    x1 = (x1 + ks[1]).astype(np.uint32)
    for i in range(5):
        for r in rots[i % 2]:
            x0 = (x0 + x1).astype(np.uint32)
            x1 = ((x1 << np.uint32(r)) | (x1 >> np.uint32(32 - r))).astype(np.uint32)
            x1 = x1 ^ x0
        x0 = (x0 + ks[(i + 1) % 3]).astype(np.uint32)
        x1 = (x1 + ks[(i + 2) % 3] + np.uint32(i + 1)).astype(np.uint32)
    return x0, x1


def _sample_key():
    o0, o1 = _np_threefry2x32(np.uint32(0), np.uint32(1),
                              np.array([0], np.uint32), np.array([7], np.uint32))
    return int(o0[0]), int(o1[0])


_K0, _K1 = _sample_key()


def _phase_a(xb_ref, xa_ref, lw_ref, gmax_ref):
    i = pl.program_id(0)
    xb = xb_ref[...]
    xa = xa_ref[...]
    sim = jax.lax.dot_general(
        xb, xa, (((1,), (1,)), ((), ())),
        preferred_element_type=jnp.float32)
    r = jax.lax.broadcasted_iota(jnp.int32, sim.shape, 0) + i * _RA
    c = jax.lax.broadcasted_iota(jnp.int32, sim.shape, 1)
    d2 = 2.0 - 2.0 * sim + jnp.where(r == c, 1.0, 0.0)
    dist = jnp.sqrt(d2)
    distance = jnp.maximum(dist, _CUTOFF)
    lw = (-62.0) * jnp.log(distance) - 30.5 * jnp.log(
        jnp.maximum(1.0 - 0.25 * (distance * distance), 1e-08))
    bmax = jnp.max(lw)

    @pl.when(i == 0)
    def _():
        gmax_ref[0, 0] = bmax

    @pl.when(i > 0)
    def _():
        gmax_ref[0, 0] = jnp.maximum(gmax_ref[0, 0], bmax)

    keep = jnp.logical_and((r // _K) != (c // _K), distance < _NLC)
    lw_ref[...] = jnp.where(keep, lw, -jnp.inf)


def _threefry_bits(flat):
    k0 = np.uint32(_K0)
    k1 = np.uint32(_K1)
    k2 = np.uint32(_K0) ^ np.uint32(_K1) ^ np.uint32(0x1BD11BDA)
    ks = (k0, k1, k2)
    rots = ((13, 15, 26, 6), (17, 29, 16, 24))
    x0 = jnp.full(flat.shape, k0, jnp.uint32)
    x1 = flat + k1
    for i in range(5):
        for r in rots[i % 2]:
            x0 = x0 + x1
            x1 = (x1 << np.uint32(r)) | (x1 >> np.uint32(32 - r))
            x1 = x1 ^ x0
        x0 = x0 + ks[(i + 1) % 3]
        x1 = x1 + (ks[(i + 2) % 3] + np.uint32(i + 1))
    return x0 ^ x1


def _gumbel(flat):
    bits = _threefry_bits(flat)
    fb = (bits >> np.uint32(9)) | np.uint32(0x3F800000)
    u = jax.lax.bitcast_convert_type(fb, jnp.float32) - 1.0
    u = u + _TINY
    u = jnp.maximum(_TINY, u)
    return -jnp.log(-jnp.log(u))


def _phase_b(gmax_ref, lw_ref, anc_ref, pos_ref, neg_ref):
    i = pl.program_id(0)
    lw = lw_ref[...]
    m = gmax_ref[0, 0]
    w = jnp.exp(lw - m)
    wsum = jnp.sum(w, axis=1, keepdims=True)
    r = jax.lax.broadcasted_iota(jnp.int32, lw.shape, 0) + i * _RB
    c = jax.lax.broadcasted_iota(jnp.int32, lw.shape, 1)
    maskf = jnp.where((r // _K) != (c // _K), 1.0, 0.0)
    uni = maskf * (1.0 / (_N - _K))
    probs = jnp.where(wsum > 0, w / (wsum + 1e-08), uni)
    logits = jnp.log(probs + 1e-20)

    flat_rc = (r * _N + c).astype(jnp.uint32)
    cols = []
    for s in range(_S):
        flat = flat_rc + np.uint32(s * _N * _N)
        val = logits + _gumbel(flat)
        vmax = jnp.max(val, axis=1, keepdims=True)
        idx = jnp.min(jnp.where(val == vmax, c, _N), axis=1, keepdims=True)
        cols.append(idx)
    neg_ref[...] = jnp.concatenate(cols, axis=1)

    rs = jax.lax.broadcasted_iota(jnp.int32, (_RB, _S), 0) + i * _RB
    offs = jax.lax.broadcasted_iota(jnp.int32, (_RB, _S), 1)
    self_off = rs % _K
    pos = (rs // _K) * _K + offs + jnp.where(offs >= self_off, 1, 0)
    pos_ref[...] = pos
    anc_ref[...] = rs


def kernel(embeddings):
    x = embeddings
    lw, gmax = pl.pallas_call(
        _phase_a,
        grid=(_N // _RA,),
        in_specs=[pl.BlockSpec((_RA, _D), lambda i: (i, 0)),
                  pl.BlockSpec((_N, _D), lambda i: (0, 0))],
        out_specs=[pl.BlockSpec((_RA, _N), lambda i: (i, 0)),
                   pl.BlockSpec((1, 1), lambda i: (0, 0))],
        out_shape=[jax.ShapeDtypeStruct((_N, _N), jnp.float32),
                   jax.ShapeDtypeStruct((1, 1), jnp.float32)],
        interpret=False,
    )(x, x)
    anc, pos, neg = pl.pallas_call(
        _phase_b,
        grid=(_N // _RB,),
        in_specs=[pl.BlockSpec((1, 1), lambda i: (0, 0)),
                  pl.BlockSpec((_RB, _N), lambda i: (i, 0))],
        out_specs=[pl.BlockSpec((_RB, _S), lambda i: (i, 0)),
                   pl.BlockSpec((_RB, _S), lambda i: (i, 0)),
                   pl.BlockSpec((_RB, _S), lambda i: (i, 0))],
        out_shape=[jax.ShapeDtypeStruct((_N, _S), jnp.int32),
                   jax.ShapeDtypeStruct((_N, _S), jnp.int32),
                   jax.ShapeDtypeStruct((_N, _S), jnp.int32)],
        interpret=False,
    )(gmax, lw)
    triplets = jnp.stack(
        [anc.reshape(-1), pos.reshape(-1), neg.reshape(-1)], axis=1)
    return triplets.astype(jnp.int64)


# speedup vs baseline: 1.3991x; 1.3991x over previous
import functools

import numpy as np
import jax
import jax.numpy as jnp
from jax.experimental import pallas as pl
from jax.experimental.pallas import tpu as pltpu
from jax.experimental.pallas import tpu_sc as plsc

_N = 4096
_D = 64
_K = 4
_S = _K - 1
_RA = 256
_RB = 128
_CUTOFF = 0.5
_NLC = 1.4
_TINY = np.float32(np.finfo(np.float32).tiny)


def _np_threefry2x32(k0, k1, x0, x1):
    ks = (np.uint32(k0), np.uint32(k1),
          np.uint32(k0) ^ np.uint32(k1) ^ np.uint32(0x1BD11BDA))
    rots = ((13, 15, 26, 6), (17, 29, 16, 24))
    x0 = (x0 + ks[0]).astype(np.uint32)
    x1 = (x1 + ks[1]).astype(np.uint32)
    for i in range(5):
        for r in rots[i % 2]:
            x0 = (x0 + x1).astype(np.uint32)
            x1 = ((x1 << np.uint32(r)) | (x1 >> np.uint32(32 - r))).astype(np.uint32)
            x1 = x1 ^ x0
        x0 = (x0 + ks[(i + 1) % 3]).astype(np.uint32)
        x1 = (x1 + ks[(i + 2) % 3] + np.uint32(i + 1)).astype(np.uint32)
    return x0, x1


def _sample_key():
    o0, o1 = _np_threefry2x32(np.uint32(0), np.uint32(1),
                              np.array([0], np.uint32), np.array([7], np.uint32))
    return int(o0[0]), int(o1[0])


_K0, _K1 = _sample_key()


def _phase_a(xb_ref, xa_ref, lw_ref, gmax_ref):
    i = pl.program_id(0)
    sim = jax.lax.dot_general(
        xb_ref[...], xa_ref[...], (((1,), (1,)), ((), ())),
        preferred_element_type=jnp.float32)
    r = jax.lax.broadcasted_iota(jnp.int32, sim.shape, 0) + i * _RA
    c = jax.lax.broadcasted_iota(jnp.int32, sim.shape, 1)
    d2 = 2.0 - 2.0 * sim + jnp.where(r == c, 1.0, 0.0)
    dist = jnp.sqrt(d2)
    distance = jnp.maximum(dist, _CUTOFF)
    lw = (-62.0) * jnp.log(distance) - 30.5 * jnp.log(
        jnp.maximum(1.0 - 0.25 * (distance * distance), 1e-08))
    bmax = jnp.max(lw, axis=(0, 1), keepdims=True)

    @pl.when(i == 0)
    def _():
        gmax_ref[...] = bmax

    @pl.when(i > 0)
    def _():
        gmax_ref[...] = jnp.maximum(gmax_ref[...], bmax)

    keep = jnp.logical_and((r // _K) != (c // _K), distance < _NLC)
    lw_ref[...] = jnp.where(keep, lw, -jnp.inf)


_CW = 128
_RH = 128
_NCHUNK = _N // _CW


def _threefry_bits(x1):
    k0 = np.uint32(_K0)
    k1 = np.uint32(_K1)
    k2 = np.uint32(_K0) ^ np.uint32(_K1) ^ np.uint32(0x1BD11BDA)
    ks = (k0, k1, k2)
    rots = ((13, 15, 26, 6), (17, 29, 16, 24))
    x0 = jnp.full(x1.shape, k0, jnp.uint32)
    for i in range(5):
        for r in rots[i % 2]:
            x0 = x0 + x1
            x1 = (x1 << np.uint32(r)) | (x1 >> np.uint32(32 - r))
            x1 = x1 ^ x0
        x0 = x0 + ks[(i + 1) % 3]
        x1 = x1 + (ks[(i + 2) % 3] + np.uint32(i + 1))
    return x0 ^ x1


def _neg_log_gumbel(x1):
    bits = _threefry_bits(x1)
    fb = (bits >> np.uint32(9)) | np.uint32(0x3F800000)
    u = jax.lax.bitcast_convert_type(fb, jnp.float32) - 1.0
    u = u + _TINY
    return jnp.log(-jnp.log(u))


_SC_NW = 32
_SC_TAIL = 128
_SC_N = _N - _SC_TAIL
_SC_ROWS = _SC_N // _SC_NW


def _sc_bits2(out_ref, buf_ref, sem0, sem1):
    wid = jax.lax.axis_index("s") * 2 + jax.lax.axis_index("c")
    row0 = wid * _SC_ROWS
    lanes = jnp.arange(16, dtype=jnp.int32)
    sems = (sem0, sem1)
    cbase = np.uint32((2 * _N * _N + _K1) & 0xFFFFFFFF)

    def pair_body(p, carry):
        for b in range(2):
            row = row0 + p * 2 + b

            @pl.when(p > 0)
            def _(b=b, row=row):
                pltpu.make_async_copy(
                    buf_ref.at[b], out_ref.at[row - 2], sems[b]).wait()

            def vec_body(v, c, row=row, b=b):
                flat = (row * _N + v * 16 + lanes).astype(jnp.uint32)
                buf_ref[b, pl.ds(v * 16, 16)] = _threefry_bits(flat + cbase)
                return c

            jax.lax.fori_loop(0, _N // 16, vec_body, 0)
            pltpu.make_async_copy(
                buf_ref.at[b], out_ref.at[row], sems[b]).start()
        return carry

    jax.lax.fori_loop(0, _SC_ROWS // 2, pair_body, 0)
    last = row0 + _SC_ROWS - 1
    pltpu.make_async_copy(buf_ref.at[0], out_ref.at[last - 1], sems[0]).wait()
    pltpu.make_async_copy(buf_ref.at[1], out_ref.at[last], sems[1]).wait()


def _sc_bits2_call():
    fn = functools.partial(
        pl.kernel,
        out_type=jax.ShapeDtypeStruct((_SC_N, _N), jnp.uint32),
        mesh=plsc.VectorSubcoreMesh(core_axis_name="c", subcore_axis_name="s",
                                    num_cores=2),
        scratch_types=[pltpu.VMEM((2, _N), jnp.uint32),
                       pltpu.SemaphoreType.DMA,
                       pltpu.SemaphoreType.DMA],
    )(_sc_bits2)
    return fn()


def _phase_b1(gmax_ref, lw_ref, logits_ref, neg01_ref):
    i = pl.program_id(0)
    lw = lw_ref[...]
    m = gmax_ref[...]
    w = jnp.exp(lw - m)
    wsum = jnp.sum(w, axis=1, keepdims=True)
    r = jax.lax.broadcasted_iota(jnp.int32, lw.shape, 0) + i * _RB
    c = jax.lax.broadcasted_iota(jnp.int32, lw.shape, 1)
    maskf = jnp.where((r // _K) != (c // _K), 1.0, 0.0)
    uni = maskf * (1.0 / (_N - _K))
    probs = jnp.where(wsum > 0, w / (wsum + 1e-08), uni)
    logits_ref[...] = jnp.log(probs + 1e-20)

    def draw(s):
        parts = []
        lane = jax.lax.broadcasted_iota(jnp.int32, (_RH, _CW), 1)
        for h in range(_RB // _RH):
            rch = (jax.lax.broadcasted_iota(jnp.int32, (_RH, _CW), 0)
                   + (i * _RB + h * _RH))
            base = (rch * _N + lane).astype(jnp.uint32)

            def body(j, carry, base=base, h=h, s=s):
                run_val, run_idx = carry
                off = (j * _CW).astype(jnp.uint32) + np.uint32(
                    (s * _N * _N + _K1) & 0xFFFFFFFF)
                val = (logits_ref[pl.ds(h * _RH, _RH), pl.ds(j * _CW, _CW)]
                       - _neg_log_gumbel(base + off))
                colv = lane + j * _CW
                upd = val > run_val
                return jnp.maximum(run_val, val), jnp.where(upd, colv, run_idx)

            run_val = jnp.full((_RH, _CW), -jnp.inf, jnp.float32)
            run_idx = jnp.zeros((_RH, _CW), jnp.int32)
            run_val, run_idx = jax.lax.fori_loop(
                0, _NCHUNK, body, (run_val, run_idx), unroll=16)
            vmax = jnp.max(run_val, axis=1, keepdims=True)
            parts.append(jnp.min(jnp.where(run_val == vmax, run_idx, _N),
                                 axis=1, keepdims=True))
        return jnp.concatenate(parts, axis=0)

    cols = [draw(0), draw(1), jnp.zeros((_RB, 1), jnp.int32)]
    neg01_ref[...] = jnp.concatenate(cols, axis=1)

    @pl.when(i >= _SC_N // _RB)
    def _():
        neg01_ref[:, 2:3] = draw(2)


def _phase_b2(logits_ref, bits_ref, neg01_ref, out_ref):
    i = pl.program_id(0)
    rh_n = _RB // _RH
    lane = jax.lax.broadcasted_iota(jnp.int32, (_RH, _CW), 1)
    parts = []
    for h in range(rh_n):
        def body(j, carry, h=h):
            run_val, run_idx = carry
            bits = bits_ref[pl.ds(h * _RH, _RH), pl.ds(j * _CW, _CW)]
            fb = (bits >> np.uint32(9)) | np.uint32(0x3F800000)
            u = jax.lax.bitcast_convert_type(fb, jnp.float32) - 1.0
            u = u + _TINY
            val = (logits_ref[pl.ds(h * _RH, _RH), pl.ds(j * _CW, _CW)]
                   - jnp.log(-jnp.log(u)))
            colv = lane + j * _CW
            upd = val > run_val
            return jnp.maximum(run_val, val), jnp.where(upd, colv, run_idx)

        run_val = jnp.full((_RH, _CW), -jnp.inf, jnp.float32)
        run_idx = jnp.zeros((_RH, _CW), jnp.int32)
        run_val, run_idx = jax.lax.fori_loop(
            0, _NCHUNK, body, (run_val, run_idx), unroll=8)
        vmax = jnp.max(run_val, axis=1, keepdims=True)
        parts.append(jnp.min(jnp.where(run_val == vmax, run_idx, _N),
                             axis=1, keepdims=True))
    neg2 = jnp.concatenate(parts, axis=0)
    neg01 = neg01_ref[...]
    neg2 = jnp.where(i >= _SC_N // _RB, neg01[:, 2:3], neg2)

    anc = jax.lax.broadcasted_iota(jnp.int32, (_RB, 1), 0) + i * _RB
    self_off = anc % _K
    blk = (anc // _K) * _K
    negs = [neg01[:, 0:1], neg01[:, 1:2], neg2]
    nine = []
    for s in range(_S):
        pos = blk + s + jnp.where(s >= self_off, 1, 0)
        nine.extend([anc, pos, negs[s]])
    out_ref[...] = jnp.concatenate(nine, axis=1)


def kernel(embeddings):
    x = embeddings
    lw, gmax = pl.pallas_call(
        _phase_a,
        grid=(_N // _RA,),
        in_specs=[pl.BlockSpec((_RA, _D), lambda i: (i, 0)),
                  pl.BlockSpec((_N, _D), lambda i: (0, 0))],
        out_specs=[pl.BlockSpec((_RA, _N), lambda i: (i, 0)),
                   pl.BlockSpec((1, 1), lambda i: (0, 0))],
        out_shape=[jax.ShapeDtypeStruct((_N, _N), jnp.float32),
                   jax.ShapeDtypeStruct((1, 1), jnp.float32)],
        interpret=False,
    )(x, x)
    bits2 = _sc_bits2_call()
    logits, neg01 = pl.pallas_call(
        _phase_b1,
        grid=(_N // _RB,),
        in_specs=[pl.BlockSpec((1, 1), lambda i: (0, 0)),
                  pl.BlockSpec((_RB, _N), lambda i: (i, 0))],
        out_specs=[pl.BlockSpec((_RB, _N), lambda i: (i, 0)),
                   pl.BlockSpec((_RB, 3), lambda i: (i, 0))],
        out_shape=[jax.ShapeDtypeStruct((_N, _N), jnp.float32),
                   jax.ShapeDtypeStruct((_N, 3), jnp.int32)],
        interpret=False,
    )(gmax, lw)
    out9 = pl.pallas_call(
        _phase_b2,
        grid=(_N // _RB,),
        in_specs=[pl.BlockSpec((_RB, _N), lambda i: (i, 0)),
                  pl.BlockSpec((_RB, _N),
                               lambda i: (jnp.minimum(i, _SC_N // _RB - 1), 0)),
                  pl.BlockSpec((_RB, 3), lambda i: (i, 0))],
        out_specs=pl.BlockSpec((_RB, 3 * _S), lambda i: (i, 0)),
        out_shape=jax.ShapeDtypeStruct((_N, 3 * _S), jnp.int32),
        interpret=False,
    )(logits, bits2, neg01)
    return out9.reshape(_N * _S, 3).astype(jnp.int64)
